# Initial kernel scaffold; baseline (speedup 1.0000x reference)
import jax
import jax.numpy as jnp
from jax.experimental import pallas as pl


def kernel(X, edge_index, edge_weight, Wz, bz, Wr, br, Wh, bh, Lzw, Lzb, Lrw, Lrb, Lhw, Lhb):
    raise NotImplementedError("write your pallas kernel here")


# speedup vs baseline: 25.4734x; 25.4734x over previous
import functools

import jax
import jax.numpy as jnp
from jax import lax
from jax.experimental import pallas as pl
from jax.experimental.pallas import tpu as pltpu
from jax.experimental.pallas import tpu_sc as plsc

N = 10000
E = 320000
D = 128

NC = 2
NS = 16
NW = NC * NS
EPW = E // NW
CE = 80
NCHUNK = EPW // CE
RPT = N // NS
ZR = 25

BN = 1024
GRID = (N + BN - 1) // BN

_sc_mesh = plsc.VectorSubcoreMesh(core_axis_name="c", subcore_axis_name="s")


def _deg_body(dst_hbm, ew_hbm, degp_hbm, dst_v, ew_v, deg_v):
    cid = lax.axis_index("c")
    sid = lax.axis_index("s")
    wid = cid * NS + sid
    base = wid * EPW

    def zbody(i, _):
        deg_v[pl.ds(i * 16, 16)] = jnp.zeros((16,), jnp.float32)
        return 0

    lax.fori_loop(0, N // 16, zbody, 0)

    pltpu.sync_copy(dst_hbm.at[pl.ds(base, EPW)], dst_v)
    pltpu.sync_copy(ew_hbm.at[pl.ds(base, EPW)], ew_v)

    def ebody(i, _):
        idx = dst_v[pl.ds(i * 16, 16)]
        w = ew_v[pl.ds(i * 16, 16)]
        plsc.addupdate_scatter(deg_v, [idx], w)
        return 0

    lax.fori_loop(0, EPW // 16, ebody, 0)
    pltpu.sync_copy(deg_v, degp_hbm.at[wid])


_deg_call = pl.kernel(
    _deg_body,
    out_type=jax.ShapeDtypeStruct((NW, N), jnp.float32),
    mesh=_sc_mesh,
    compiler_params=pltpu.CompilerParams(needs_layout_passes=False, use_tc_tiling_on_sc=False),
    scratch_types=[
        pltpu.VMEM((EPW,), jnp.int32),
        pltpu.VMEM((EPW,), jnp.float32),
        pltpu.VMEM((N,), jnp.float32),
    ],
)


def _prep_body(degp_ref, x_ref, y_ref, dinv_ref):
    deg = jnp.sum(degp_ref[...], axis=0) + 1.0
    dinv = lax.rsqrt(deg)[:, None]
    y_ref[...] = x_ref[...] * dinv
    dinv_ref[...] = dinv


_prep_call = pl.pallas_call(
    _prep_body,
    grid=(GRID,),
    in_specs=[
        pl.BlockSpec((NW, BN), lambda i: (0, i)),
        pl.BlockSpec((BN, D), lambda i: (i, 0)),
    ],
    out_specs=[
        pl.BlockSpec((BN, D), lambda i: (i, 0)),
        pl.BlockSpec((BN, 1), lambda i: (i, 0)),
    ],
    out_shape=[
        jax.ShapeDtypeStruct((N, D), jnp.float32),
        jax.ShapeDtypeStruct((N, 1), jnp.float32),
    ],
)


def _edge_body(src_hbm, dst_hbm, ew_hbm, y_hbm, accp_hbm,
               acc_sh, src_v, dst_v, ew_v, rows_v, zero_v, gsem):
    cid = lax.axis_index("c")
    sid = lax.axis_index("s")
    wid = cid * NS + sid
    rowbase = sid * RPT

    def zb(i, _):
        for k in range(8):
            zero_v[i, pl.ds(k * 16, 16)] = jnp.zeros((16,), jnp.float32)
        return 0

    lax.fori_loop(0, ZR, zb, 0)

    def zcopy(j, _):
        pltpu.sync_copy(zero_v, acc_sh.at[pl.ds(rowbase + j * ZR, ZR)])
        return 0

    lax.fori_loop(0, RPT // ZR, zcopy, 0)
    plsc.subcore_barrier()

    ebase = wid * EPW

    def chunk(c, _):
        base = ebase + c * CE
        pltpu.sync_copy(src_hbm.at[pl.ds(base, CE)], src_v)
        pltpu.sync_copy(ew_hbm.at[pl.ds(base, CE)], ew_v)
        pltpu.sync_copy(dst_hbm.at[pl.ds(base, CE)], dst_v)
        pltpu.async_copy(y_hbm.at[src_v], rows_v, gsem).wait()

        def scale(g, _):
            wv = ew_v[pl.ds(g * 16, 16)]
            for j in range(16):
                w = wv[j]
                e = g * 16 + j
                for k in range(8):
                    rows_v[e, pl.ds(k * 16, 16)] = (
                        rows_v[e, pl.ds(k * 16, 16)] * w)
            return 0

        lax.fori_loop(0, CE // 16, scale, 0)
        pltpu.sync_copy(rows_v, acc_sh.at[dst_v], add=True)
        return 0

    lax.fori_loop(0, NCHUNK, chunk, 0)
    plsc.subcore_barrier()
    pltpu.sync_copy(acc_sh.at[pl.ds(rowbase, RPT)],
                    accp_hbm.at[cid, pl.ds(rowbase, RPT)])


_edge_call = pl.kernel(
    _edge_body,
    out_type=jax.ShapeDtypeStruct((NC, N, D), jnp.float32),
    mesh=_sc_mesh,
    compiler_params=pltpu.CompilerParams(needs_layout_passes=False, use_tc_tiling_on_sc=False),
    scratch_types=[
        pltpu.VMEM_SHARED((N, D), jnp.float32),
        pltpu.VMEM((CE,), jnp.int32),
        pltpu.VMEM((CE,), jnp.int32),
        pltpu.VMEM((CE,), jnp.float32),
        pltpu.VMEM((CE, D), jnp.float32),
        pltpu.VMEM((ZR, D), jnp.float32),
        pltpu.SemaphoreType.DMA,
    ],
)


def _out_body(acc_ref, y_ref, dinv_ref, mz_ref, mh_ref, wh_ref,
              cz_ref, ch_ref, bh_ref, o_ref):
    p = dinv_ref[...] * (acc_ref[0] + acc_ref[1] + y_ref[...])
    z = jax.nn.sigmoid(
        jnp.dot(p, mz_ref[...], preferred_element_type=jnp.float32)
        + cz_ref[...])
    t = (jnp.tanh(jnp.dot(p, mh_ref[...], preferred_element_type=jnp.float32)
                  + ch_ref[...])
         + jnp.dot(p, wh_ref[...], preferred_element_type=jnp.float32)
         + bh_ref[...])
    o_ref[...] = (1.0 - z) * t


_full = lambda i: (0, 0)
_out_call = pl.pallas_call(
    _out_body,
    grid=(GRID,),
    in_specs=[
        pl.BlockSpec((NC, BN, D), lambda i: (0, i, 0)),
        pl.BlockSpec((BN, D), lambda i: (i, 0)),
        pl.BlockSpec((BN, 1), lambda i: (i, 0)),
        pl.BlockSpec((D, D), _full),
        pl.BlockSpec((D, D), _full),
        pl.BlockSpec((D, D), _full),
        pl.BlockSpec((1, D), _full),
        pl.BlockSpec((1, D), _full),
        pl.BlockSpec((1, D), _full),
    ],
    out_specs=pl.BlockSpec((BN, D), lambda i: (i, 0)),
    out_shape=jax.ShapeDtypeStruct((N, D), jnp.float32),
)


@jax.jit
def kernel(X, edge_index, edge_weight, Wz, bz, Wr, br, Wh, bh,
           Lzw, Lzb, Lrw, Lrb, Lhw, Lhb):
    src = edge_index[0]
    dst = edge_index[1]
    eye = jnp.eye(D, dtype=jnp.float32)
    Lz = Lzw[:D]
    Lh = Lhw[:D]
    Mz = Wz @ (Lz + eye)
    cz = (bz @ Lz + bz + Lzb)[None, :]
    Mh = Wh @ Lh
    ch = (bh @ Lh + Lhb)[None, :]
    bh2 = bh[None, :]

    degp = _deg_call(dst, edge_weight)
    Y, dinv = _prep_call(degp, X)
    accp = _edge_call(src, dst, edge_weight, Y)
    return _out_call(accp, Y, dinv, Mz, Mh, Wh, cz, ch, bh2)


# speedup vs baseline: 49.4646x; 1.9418x over previous
import functools

import jax
import jax.numpy as jnp
from jax import lax
from jax.experimental import pallas as pl
from jax.experimental.pallas import tpu as pltpu
from jax.experimental.pallas import tpu_sc as plsc

N = 10000
E = 320000
D = 128

NC = 2
NS = 16
NW = NC * NS
EPW = E // NW
CE = 16
NCHUNK = EPW // CE
RPT = N // NS
ZR = 25

BN = 1024
GRID = (N + BN - 1) // BN

_sc_mesh = plsc.VectorSubcoreMesh(core_axis_name="c", subcore_axis_name="s")


def _deg_body(dst_hbm, ew_hbm, degp_hbm, dst_v, ew_v, deg_v):
    cid = lax.axis_index("c")
    sid = lax.axis_index("s")
    wid = cid * NS + sid
    base = wid * EPW

    def zbody(i, _):
        deg_v[pl.ds(i * 16, 16)] = jnp.zeros((16,), jnp.float32)
        return 0

    lax.fori_loop(0, N // 16, zbody, 0)

    pltpu.sync_copy(dst_hbm.at[pl.ds(base, EPW)], dst_v)
    pltpu.sync_copy(ew_hbm.at[pl.ds(base, EPW)], ew_v)

    def ebody(i, _):
        idx = dst_v[pl.ds(i * 16, 16)]
        w = ew_v[pl.ds(i * 16, 16)]
        plsc.addupdate_scatter(deg_v, [idx], w)
        return 0

    lax.fori_loop(0, EPW // 16, ebody, 0)
    pltpu.sync_copy(deg_v, degp_hbm.at[wid])


_deg_call = pl.kernel(
    _deg_body,
    out_type=jax.ShapeDtypeStruct((NW, N), jnp.float32),
    mesh=_sc_mesh,
    compiler_params=pltpu.CompilerParams(needs_layout_passes=False, use_tc_tiling_on_sc=False),
    scratch_types=[
        pltpu.VMEM((EPW,), jnp.int32),
        pltpu.VMEM((EPW,), jnp.float32),
        pltpu.VMEM((N,), jnp.float32),
    ],
)


def _prep_body(degp_ref, x_ref, y_ref, dinv_ref):
    deg = jnp.sum(degp_ref[...], axis=0) + 1.0
    dinv = lax.rsqrt(deg)[:, None]
    y_ref[...] = x_ref[...] * dinv
    dinv_ref[...] = dinv


_prep_call = pl.pallas_call(
    _prep_body,
    grid=(GRID,),
    in_specs=[
        pl.BlockSpec((NW, BN), lambda i: (0, i)),
        pl.BlockSpec((BN, D), lambda i: (i, 0)),
    ],
    out_specs=[
        pl.BlockSpec((BN, D), lambda i: (i, 0)),
        pl.BlockSpec((BN, 1), lambda i: (i, 0)),
    ],
    out_shape=[
        jax.ShapeDtypeStruct((N, D), jnp.float32),
        jax.ShapeDtypeStruct((N, 1), jnp.float32),
    ],
)


NBUF = 5
NOUT = NCHUNK // NBUF


def _edge_body(src_hbm, dst_hbm, ew_hbm, y_hbm, accp_hbm,
               acc_sh, src_v, dst_v, ew_v,
               r0, r1, r2, r3, r4, zero_v, gsem, ssem):
    rows = (r0, r1, r2, r3, r4)
    cid = lax.axis_index("c")
    sid = lax.axis_index("s")
    wid = cid * NS + sid
    rowbase = sid * RPT

    pltpu.sync_copy(src_hbm.at[pl.ds(wid * EPW, EPW)], src_v)
    pltpu.sync_copy(ew_hbm.at[pl.ds(wid * EPW, EPW)], ew_v)
    pltpu.sync_copy(dst_hbm.at[pl.ds(wid * EPW, EPW)], dst_v)

    def zb(i, _):
        for k in range(8):
            zero_v[i, pl.ds(k * 16, 16)] = jnp.zeros((16,), jnp.float32)
        return 0

    lax.fori_loop(0, ZR, zb, 0)

    def zcopy(j, _):
        pltpu.sync_copy(zero_v, acc_sh.at[pl.ds(rowbase + j * ZR, ZR)])
        return 0

    lax.fori_loop(0, RPT // ZR, zcopy, 0)
    plsc.subcore_barrier()

    def start_gather(c, b):
        idx = src_v[pl.ds(c * CE, CE)]
        pltpu.async_copy(y_hbm.at[idx], rows[b], gsem.at[b])

    for b in range(NBUF):
        start_gather(b, b)

    def outer(g, _):
        c0 = g * NBUF
        for b in range(NBUF):
            c = c0 + b
            pltpu.make_async_copy(y_hbm.at[src_v[pl.ds(0, CE)]],
                                  rows[b], gsem.at[b]).wait()
            wv = ew_v[pl.ds(c * CE, CE)]
            for j in range(CE):
                w = wv[j]
                for k in range(8):
                    rows[b][j, pl.ds(k * 16, 16)] = (
                        rows[b][j, pl.ds(k * 16, 16)] * w)
            didx = dst_v[pl.ds(c * CE, CE)]
            pltpu.async_copy(rows[b], acc_sh.at[didx], ssem.at[b], add=True)
        for b in range(NBUF):
            c2 = c0 + b + NBUF
            pltpu.make_async_copy(rows[b], acc_sh.at[dst_v[pl.ds(0, CE)]],
                                  ssem.at[b]).wait()

            @pl.when(c2 < NCHUNK)
            def _(c2=c2, b=b):
                start_gather(c2, b)
        return 0

    lax.fori_loop(0, NOUT, outer, 0)
    plsc.subcore_barrier()
    pltpu.sync_copy(acc_sh.at[pl.ds(rowbase, RPT)],
                    accp_hbm.at[cid, pl.ds(rowbase, RPT)])


_edge_call = pl.kernel(
    _edge_body,
    out_type=jax.ShapeDtypeStruct((NC, N, D), jnp.float32),
    mesh=_sc_mesh,
    compiler_params=pltpu.CompilerParams(needs_layout_passes=False, use_tc_tiling_on_sc=False),
    scratch_types=[
        pltpu.VMEM_SHARED((N, D), jnp.float32),
        pltpu.VMEM((EPW,), jnp.int32),
        pltpu.VMEM((EPW,), jnp.int32),
        pltpu.VMEM((EPW,), jnp.float32),
        pltpu.VMEM((CE, D), jnp.float32),
        pltpu.VMEM((CE, D), jnp.float32),
        pltpu.VMEM((CE, D), jnp.float32),
        pltpu.VMEM((CE, D), jnp.float32),
        pltpu.VMEM((CE, D), jnp.float32),
        pltpu.VMEM((ZR, D), jnp.float32),
        pltpu.SemaphoreType.DMA((NBUF,)),
        pltpu.SemaphoreType.DMA((NBUF,)),
    ],
)


def _out_body(acc_ref, y_ref, dinv_ref, mz_ref, mh_ref, wh_ref,
              cz_ref, ch_ref, bh_ref, o_ref):
    p = dinv_ref[...] * (acc_ref[0] + acc_ref[1] + y_ref[...])
    z = jax.nn.sigmoid(
        jnp.dot(p, mz_ref[...], preferred_element_type=jnp.float32)
        + cz_ref[...])
    t = (jnp.tanh(jnp.dot(p, mh_ref[...], preferred_element_type=jnp.float32)
                  + ch_ref[...])
         + jnp.dot(p, wh_ref[...], preferred_element_type=jnp.float32)
         + bh_ref[...])
    o_ref[...] = (1.0 - z) * t


_full = lambda i: (0, 0)
_out_call = pl.pallas_call(
    _out_body,
    grid=(GRID,),
    in_specs=[
        pl.BlockSpec((NC, BN, D), lambda i: (0, i, 0)),
        pl.BlockSpec((BN, D), lambda i: (i, 0)),
        pl.BlockSpec((BN, 1), lambda i: (i, 0)),
        pl.BlockSpec((D, D), _full),
        pl.BlockSpec((D, D), _full),
        pl.BlockSpec((D, D), _full),
        pl.BlockSpec((1, D), _full),
        pl.BlockSpec((1, D), _full),
        pl.BlockSpec((1, D), _full),
    ],
    out_specs=pl.BlockSpec((BN, D), lambda i: (i, 0)),
    out_shape=jax.ShapeDtypeStruct((N, D), jnp.float32),
)


@jax.jit
def kernel(X, edge_index, edge_weight, Wz, bz, Wr, br, Wh, bh,
           Lzw, Lzb, Lrw, Lrb, Lhw, Lhb):
    src = edge_index[0]
    dst = edge_index[1]
    eye = jnp.eye(D, dtype=jnp.float32)
    Lz = Lzw[:D]
    Lh = Lhw[:D]
    Mz = Wz @ (Lz + eye)
    cz = (bz @ Lz + bz + Lzb)[None, :]
    Mh = Wh @ Lh
    ch = (bh @ Lh + Lhb)[None, :]
    bh2 = bh[None, :]

    degp = _deg_call(dst, edge_weight)
    Y, dinv = _prep_call(degp, X)
    accp = _edge_call(src, dst, edge_weight, Y)
    return _out_call(accp, Y, dinv, Mz, Mh, Wh, cz, ch, bh2)


# speedup vs baseline: 53.8482x; 1.0886x over previous
import functools

import jax
import jax.numpy as jnp
from jax import lax
from jax.experimental import pallas as pl
from jax.experimental.pallas import tpu as pltpu
from jax.experimental.pallas import tpu_sc as plsc

N = 10000
E = 320000
D = 128

NC = 2
NS = 16
NW = NC * NS
EPW = E // NW
CE = 80
NCHUNK = EPW // CE
RPT = N // NS
ZR = 25

BN = 1024
GRID = (N + BN - 1) // BN

_sc_mesh = plsc.VectorSubcoreMesh(core_axis_name="c", subcore_axis_name="s")


def _deg_body(dst_hbm, ew_hbm, degp_hbm, dst_v, ew_v, deg_v):
    cid = lax.axis_index("c")
    sid = lax.axis_index("s")
    wid = cid * NS + sid
    base = wid * EPW

    def zbody(i, _):
        deg_v[pl.ds(i * 16, 16)] = jnp.zeros((16,), jnp.float32)
        return 0

    lax.fori_loop(0, N // 16, zbody, 0)

    pltpu.sync_copy(dst_hbm.at[pl.ds(base, EPW)], dst_v)
    pltpu.sync_copy(ew_hbm.at[pl.ds(base, EPW)], ew_v)

    def ebody(i, _):
        idx = dst_v[pl.ds(i * 16, 16)]
        w = ew_v[pl.ds(i * 16, 16)]
        plsc.addupdate_scatter(deg_v, [idx], w)
        return 0

    lax.fori_loop(0, EPW // 16, ebody, 0)
    pltpu.sync_copy(deg_v, degp_hbm.at[wid])


_deg_call = pl.kernel(
    _deg_body,
    out_type=jax.ShapeDtypeStruct((NW, N), jnp.float32),
    mesh=_sc_mesh,
    compiler_params=pltpu.CompilerParams(needs_layout_passes=False, use_tc_tiling_on_sc=False),
    scratch_types=[
        pltpu.VMEM((EPW,), jnp.int32),
        pltpu.VMEM((EPW,), jnp.float32),
        pltpu.VMEM((N,), jnp.float32),
    ],
)


def _prep_body(degp_ref, x_ref, y_ref, dinv_ref):
    deg = jnp.sum(degp_ref[...], axis=0) + 1.0
    dinv = lax.rsqrt(deg)[:, None]
    y_ref[...] = x_ref[...] * dinv
    dinv_ref[...] = dinv


_prep_call = pl.pallas_call(
    _prep_body,
    grid=(GRID,),
    in_specs=[
        pl.BlockSpec((NW, BN), lambda i: (0, i)),
        pl.BlockSpec((BN, D), lambda i: (i, 0)),
    ],
    out_specs=[
        pl.BlockSpec((BN, D), lambda i: (i, 0)),
        pl.BlockSpec((BN, 1), lambda i: (i, 0)),
    ],
    out_shape=[
        jax.ShapeDtypeStruct((N, D), jnp.float32),
        jax.ShapeDtypeStruct((N, 1), jnp.float32),
    ],
)


NBUF = 2
NPAIR = NCHUNK // NBUF


def _edge_body(src_hbm, dstr_hbm, ew_hbm, y_hbm, accp_hbm,
               acc_sh, src_v, dst_v, ew_v, r0, r1, gsem, ssem):
    rows = (r0, r1)
    cid = lax.axis_index("c")
    sid = lax.axis_index("s")
    wid = cid * NS + sid
    rowbase = sid * RPT

    pltpu.sync_copy(src_hbm.at[pl.ds(wid * EPW, EPW)], src_v)
    pltpu.sync_copy(ew_hbm.at[pl.ds(wid * EPW, EPW)], ew_v)
    pltpu.sync_copy(dstr_hbm.at[pl.ds(wid * NCHUNK, NCHUNK)], dst_v)

    def zb(i, _):
        for k in range(8):
            r0[i, pl.ds(k * 16, 16)] = jnp.zeros((16,), jnp.float32)
        return 0

    lax.fori_loop(0, CE, zb, 0)

    def zcopy(j, _):
        pltpu.sync_copy(r0, acc_sh.at[pl.ds(rowbase + j * CE, CE)])
        return 0

    lax.fori_loop(0, RPT // CE, zcopy, 0)
    pltpu.sync_copy(r0.at[pl.ds(0, RPT % CE)],
                    acc_sh.at[pl.ds(rowbase + (RPT // CE) * CE, RPT % CE)])

    def start_gather(c, b):
        pltpu.async_copy(y_hbm.at[src_v.at[pl.ds(c * CE, CE)]],
                         rows[b], gsem.at[b])

    def wait_gather(b):
        pltpu.make_async_copy(y_hbm.at[src_v.at[pl.ds(0, CE)]],
                              rows[b], gsem.at[b]).wait()

    def scale(c, b):
        def sg(g2, _):
            wv = ew_v[pl.ds(c * CE + g2 * 16, 16)]
            for j in range(16):
                w = wv[j]
                for k in range(8):
                    rows[b][g2 * 16 + j, pl.ds(k * 16, 16)] = (
                        rows[b][g2 * 16 + j, pl.ds(k * 16, 16)] * w)
            return 0

        lax.fori_loop(0, CE // 16, sg, 0)

    def start_scatter(c, b):
        pltpu.async_copy(rows[b], acc_sh.at[dst_v.at[c]], ssem.at[b],
                         add=True)

    def wait_scatter(b):
        pltpu.make_async_copy(rows[b], acc_sh.at[dst_v.at[0]],
                              ssem.at[b]).wait()

    for b in range(NBUF):
        start_gather(b, b)
    plsc.subcore_barrier()

    def outer(g, _):
        c0 = g * NBUF
        for b in range(NBUF):
            wait_gather(b)
            scale(c0 + b, b)
            start_scatter(c0 + b, b)
        for b in range(NBUF):
            c2 = c0 + b + NBUF
            wait_scatter(b)

            @pl.when(c2 < NCHUNK)
            def _(c2=c2, b=b):
                start_gather(c2, b)
        return 0

    lax.fori_loop(0, NPAIR, outer, 0)
    wait_gather(0)
    scale(NCHUNK - 1, 0)
    start_scatter(NCHUNK - 1, 0)
    wait_scatter(0)

    plsc.subcore_barrier()
    pltpu.sync_copy(acc_sh.at[pl.ds(rowbase, RPT)],
                    accp_hbm.at[cid, pl.ds(rowbase, RPT)])


_edge_call = pl.kernel(
    _edge_body,
    out_type=jax.ShapeDtypeStruct((NC, N, D), jnp.float32),
    mesh=_sc_mesh,
    compiler_params=pltpu.CompilerParams(needs_layout_passes=False, use_tc_tiling_on_sc=False),
    scratch_types=[
        pltpu.VMEM_SHARED((N, D), jnp.float32),
        pltpu.VMEM((EPW,), jnp.int32),
        pltpu.VMEM((NCHUNK, CE), jnp.int32),
        pltpu.VMEM((EPW,), jnp.float32),
        pltpu.VMEM((CE, D), jnp.float32),
        pltpu.VMEM((CE, D), jnp.float32),
        pltpu.SemaphoreType.DMA((NBUF,)),
        pltpu.SemaphoreType.DMA((NBUF,)),
    ],
)


def _out_body(acc_ref, y_ref, dinv_ref, mz_ref, mh_ref, wh_ref,
              cz_ref, ch_ref, bh_ref, o_ref):
    p = dinv_ref[...] * (acc_ref[0] + acc_ref[1] + y_ref[...])
    z = jax.nn.sigmoid(
        jnp.dot(p, mz_ref[...], preferred_element_type=jnp.float32)
        + cz_ref[...])
    t = (jnp.tanh(jnp.dot(p, mh_ref[...], preferred_element_type=jnp.float32)
                  + ch_ref[...])
         + jnp.dot(p, wh_ref[...], preferred_element_type=jnp.float32)
         + bh_ref[...])
    o_ref[...] = (1.0 - z) * t


_full = lambda i: (0, 0)
_out_call = pl.pallas_call(
    _out_body,
    grid=(GRID,),
    in_specs=[
        pl.BlockSpec((NC, BN, D), lambda i: (0, i, 0)),
        pl.BlockSpec((BN, D), lambda i: (i, 0)),
        pl.BlockSpec((BN, 1), lambda i: (i, 0)),
        pl.BlockSpec((D, D), _full),
        pl.BlockSpec((D, D), _full),
        pl.BlockSpec((D, D), _full),
        pl.BlockSpec((1, D), _full),
        pl.BlockSpec((1, D), _full),
        pl.BlockSpec((1, D), _full),
    ],
    out_specs=pl.BlockSpec((BN, D), lambda i: (i, 0)),
    out_shape=jax.ShapeDtypeStruct((N, D), jnp.float32),
)


@jax.jit
def kernel(X, edge_index, edge_weight, Wz, bz, Wr, br, Wh, bh,
           Lzw, Lzb, Lrw, Lrb, Lhw, Lhb):
    src = edge_index[0]
    dst = edge_index[1]
    eye = jnp.eye(D, dtype=jnp.float32)
    Lz = Lzw[:D]
    Lh = Lhw[:D]
    Mz = Wz @ (Lz + eye)
    cz = (bz @ Lz + bz + Lzb)[None, :]
    Mh = Wh @ Lh
    ch = (bh @ Lh + Lhb)[None, :]
    bh2 = bh[None, :]

    degp = _deg_call(dst, edge_weight)
    Y, dinv = _prep_call(degp, X)
    dstr = dst.reshape(E // CE, CE)
    accp = _edge_call(src, dstr, edge_weight, Y)
    return _out_call(accp, Y, dinv, Mz, Mh, Wh, cz, ch, bh2)


# speedup vs baseline: 58.8325x; 1.0926x over previous
import functools

import jax
import jax.numpy as jnp
from jax import lax
from jax.experimental import pallas as pl
from jax.experimental.pallas import tpu as pltpu
from jax.experimental.pallas import tpu_sc as plsc

N = 10000
E = 320000
D = 128

NC = 2
NS = 16
NW = NC * NS
EPW = E // NW
CE = 80
NCHUNK = EPW // CE
RPT = N // NS
ZR = 25

BN = 1024
GRID = (N + BN - 1) // BN

_sc_mesh = plsc.VectorSubcoreMesh(core_axis_name="c", subcore_axis_name="s")


def _deg_body(dst_hbm, ew_hbm, degp_hbm, dst_v, ew_v, deg_v):
    cid = lax.axis_index("c")
    sid = lax.axis_index("s")
    wid = cid * NS + sid
    base = wid * EPW

    def zbody(i, _):
        deg_v[pl.ds(i * 16, 16)] = jnp.zeros((16,), jnp.float32)
        return 0

    lax.fori_loop(0, N // 16, zbody, 0)

    pltpu.sync_copy(dst_hbm.at[pl.ds(base, EPW)], dst_v)
    pltpu.sync_copy(ew_hbm.at[pl.ds(base, EPW)], ew_v)

    def ebody(i, _):
        idx = dst_v[pl.ds(i * 16, 16)]
        w = ew_v[pl.ds(i * 16, 16)]
        plsc.addupdate_scatter(deg_v, [idx], w)
        return 0

    lax.fori_loop(0, EPW // 16, ebody, 0)
    pltpu.sync_copy(deg_v, degp_hbm.at[wid])


_deg_call = pl.kernel(
    _deg_body,
    out_type=jax.ShapeDtypeStruct((NW, N), jnp.float32),
    mesh=_sc_mesh,
    compiler_params=pltpu.CompilerParams(needs_layout_passes=False, use_tc_tiling_on_sc=False),
    scratch_types=[
        pltpu.VMEM((EPW,), jnp.int32),
        pltpu.VMEM((EPW,), jnp.float32),
        pltpu.VMEM((N,), jnp.float32),
    ],
)


def _prep_body(degp_ref, x_ref, y_ref, dinv_ref):
    deg = jnp.sum(degp_ref[...], axis=0) + 1.0
    dinv = lax.rsqrt(deg)[:, None]
    y_ref[...] = x_ref[...] * dinv
    dinv_ref[...] = dinv


_prep_call = pl.pallas_call(
    _prep_body,
    grid=(GRID,),
    in_specs=[
        pl.BlockSpec((NW, BN), lambda i: (0, i)),
        pl.BlockSpec((BN, D), lambda i: (i, 0)),
    ],
    out_specs=[
        pl.BlockSpec((BN, D), lambda i: (i, 0)),
        pl.BlockSpec((BN, 1), lambda i: (i, 0)),
    ],
    out_shape=[
        jax.ShapeDtypeStruct((N, D), jnp.float32),
        jax.ShapeDtypeStruct((N, 1), jnp.float32),
    ],
)


NBUF = 3
NFULL = NCHUNK - (NCHUNK % NBUF)


def _edge_body(src_hbm, dstr_hbm, ew_hbm, y_hbm, accp_hbm,
               acc_sh, src_v, dstb, ewb, r0, r1, r2, gsem, ssem, isem):
    rows = (r0, r1, r2)
    cid = lax.axis_index("c")
    sid = lax.axis_index("s")
    wid = cid * NS + sid
    rowbase = sid * RPT

    pltpu.sync_copy(src_hbm.at[pl.ds(wid * EPW, EPW)], src_v)

    def zb(i, _):
        for k in range(8):
            r0[i, pl.ds(k * 16, 16)] = jnp.zeros((16,), jnp.float32)
        return 0

    lax.fori_loop(0, CE, zb, 0)

    def zcopy(j, _):
        pltpu.sync_copy(r0, acc_sh.at[pl.ds(rowbase + j * CE, CE)])
        return 0

    lax.fori_loop(0, RPT // CE, zcopy, 0)
    pltpu.sync_copy(r0.at[pl.ds(0, RPT % CE)],
                    acc_sh.at[pl.ds(rowbase + (RPT // CE) * CE, RPT % CE)])

    def start_gather(c, b):
        pltpu.async_copy(y_hbm.at[src_v.at[pl.ds(c * CE, CE)]],
                         rows[b], gsem.at[b])

    def start_idx(c, b):
        pltpu.async_copy(dstr_hbm.at[wid * NCHUNK + c], dstb.at[b],
                         isem.at[b])
        pltpu.async_copy(ew_hbm.at[pl.ds(wid * EPW + c * CE, CE)],
                         ewb.at[b], isem.at[b])

    def wait_idx(b):
        pltpu.make_async_copy(dstr_hbm.at[0], dstb.at[b], isem.at[b]).wait()
        pltpu.make_async_copy(ew_hbm.at[pl.ds(0, CE)], ewb.at[b],
                              isem.at[b]).wait()

    def wait_gather(b):
        pltpu.make_async_copy(y_hbm.at[src_v.at[pl.ds(0, CE)]],
                              rows[b], gsem.at[b]).wait()

    def scale(b):
        def sg(g2, _):
            wv = ewb[b, pl.ds(g2 * 16, 16)]
            for j in range(16):
                w = wv[j]
                for k in range(8):
                    rows[b][g2 * 16 + j, pl.ds(k * 16, 16)] = (
                        rows[b][g2 * 16 + j, pl.ds(k * 16, 16)] * w)
            return 0

        lax.fori_loop(0, CE // 16, sg, 0)

    def start_scatter(b):
        pltpu.async_copy(rows[b], acc_sh.at[dstb.at[b]], ssem.at[b],
                         add=True)

    def wait_scatter(b):
        pltpu.make_async_copy(rows[b], acc_sh.at[dstb.at[0]],
                              ssem.at[b]).wait()

    for b in range(NBUF):
        start_idx(b, b)
        start_gather(b, b)
    plsc.subcore_barrier()

    def step(c, b):
        wait_gather(b)
        wait_idx(b)
        scale(b)
        start_scatter(b)

    def refill(c2, b):
        wait_scatter(b)

        @pl.when(c2 < NCHUNK)
        def _(c2=c2, b=b):
            start_idx(c2, b)
            start_gather(c2, b)

    def outer(g, _):
        c0 = g * NBUF
        for b in range(NBUF):
            step(c0 + b, b)
        for b in range(NBUF):
            refill(c0 + b + NBUF, b)
        return 0

    lax.fori_loop(0, NFULL // NBUF, outer, 0)
    for i in range(NCHUNK % NBUF):
        step(NFULL + i, i)
    for i in range(NCHUNK % NBUF):
        wait_scatter(i)

    plsc.subcore_barrier()
    pltpu.sync_copy(acc_sh.at[pl.ds(rowbase, RPT)],
                    accp_hbm.at[cid, pl.ds(rowbase, RPT)])


_edge_call = pl.kernel(
    _edge_body,
    out_type=jax.ShapeDtypeStruct((NC, N, D), jnp.float32),
    mesh=_sc_mesh,
    compiler_params=pltpu.CompilerParams(needs_layout_passes=False, use_tc_tiling_on_sc=False),
    scratch_types=[
        pltpu.VMEM_SHARED((N, D), jnp.float32),
        pltpu.VMEM((EPW,), jnp.int32),
        pltpu.VMEM((NBUF, CE), jnp.int32),
        pltpu.VMEM((NBUF, CE), jnp.float32),
        pltpu.VMEM((CE, D), jnp.float32),
        pltpu.VMEM((CE, D), jnp.float32),
        pltpu.VMEM((CE, D), jnp.float32),
        pltpu.SemaphoreType.DMA((NBUF,)),
        pltpu.SemaphoreType.DMA((NBUF,)),
        pltpu.SemaphoreType.DMA((NBUF,)),
    ],
)


def _out_body(acc_ref, y_ref, dinv_ref, mz_ref, mh_ref, wh_ref,
              cz_ref, ch_ref, bh_ref, o_ref):
    p = dinv_ref[...] * (acc_ref[0] + acc_ref[1] + y_ref[...])
    z = jax.nn.sigmoid(
        jnp.dot(p, mz_ref[...], preferred_element_type=jnp.float32)
        + cz_ref[...])
    t = (jnp.tanh(jnp.dot(p, mh_ref[...], preferred_element_type=jnp.float32)
                  + ch_ref[...])
         + jnp.dot(p, wh_ref[...], preferred_element_type=jnp.float32)
         + bh_ref[...])
    o_ref[...] = (1.0 - z) * t


_full = lambda i: (0, 0)
_out_call = pl.pallas_call(
    _out_body,
    grid=(GRID,),
    in_specs=[
        pl.BlockSpec((NC, BN, D), lambda i: (0, i, 0)),
        pl.BlockSpec((BN, D), lambda i: (i, 0)),
        pl.BlockSpec((BN, 1), lambda i: (i, 0)),
        pl.BlockSpec((D, D), _full),
        pl.BlockSpec((D, D), _full),
        pl.BlockSpec((D, D), _full),
        pl.BlockSpec((1, D), _full),
        pl.BlockSpec((1, D), _full),
        pl.BlockSpec((1, D), _full),
    ],
    out_specs=pl.BlockSpec((BN, D), lambda i: (i, 0)),
    out_shape=jax.ShapeDtypeStruct((N, D), jnp.float32),
)


@jax.jit
def kernel(X, edge_index, edge_weight, Wz, bz, Wr, br, Wh, bh,
           Lzw, Lzb, Lrw, Lrb, Lhw, Lhb):
    src = edge_index[0]
    dst = edge_index[1]
    eye = jnp.eye(D, dtype=jnp.float32)
    Lz = Lzw[:D]
    Lh = Lhw[:D]
    Mz = Wz @ (Lz + eye)
    cz = (bz @ Lz + bz + Lzb)[None, :]
    Mh = Wh @ Lh
    ch = (bh @ Lh + Lhb)[None, :]
    bh2 = bh[None, :]

    degp = _deg_call(dst, edge_weight)
    Y, dinv = _prep_call(degp, X)
    dstr = dst.reshape(E // CE, CE)
    accp = _edge_call(src, dstr, edge_weight, Y)
    return _out_call(accp, Y, dinv, Mz, Mh, Wh, cz, ch, bh2)


# speedup vs baseline: 60.1413x; 1.0222x over previous
import functools

import jax
import jax.numpy as jnp
from jax import lax
from jax.experimental import pallas as pl
from jax.experimental.pallas import tpu as pltpu
from jax.experimental.pallas import tpu_sc as plsc

N = 10000
E = 320000
D = 128

NC = 2
NS = 16
NW = NC * NS
EPW = E // NW
CE = 80
NCHUNK = EPW // CE
RPT = N // NS
ZR = 25

BN = 1024
GRID = (N + BN - 1) // BN

_sc_mesh = plsc.VectorSubcoreMesh(core_axis_name="c", subcore_axis_name="s")


def _deg_body(dst_hbm, ew_hbm, degp_hbm, dst_v, ew_v, deg_v):
    cid = lax.axis_index("c")
    sid = lax.axis_index("s")
    wid = cid * NS + sid
    base = wid * EPW

    def zbody(i, _):
        deg_v[pl.ds(i * 16, 16)] = jnp.zeros((16,), jnp.float32)
        return 0

    lax.fori_loop(0, N // 16, zbody, 0)

    pltpu.sync_copy(dst_hbm.at[pl.ds(base, EPW)], dst_v)
    pltpu.sync_copy(ew_hbm.at[pl.ds(base, EPW)], ew_v)

    def ebody(i, _):
        idx = dst_v[pl.ds(i * 16, 16)]
        w = ew_v[pl.ds(i * 16, 16)]
        plsc.addupdate_scatter(deg_v, [idx], w)
        return 0

    lax.fori_loop(0, EPW // 16, ebody, 0)
    pltpu.sync_copy(deg_v, degp_hbm.at[wid])


_deg_call = pl.kernel(
    _deg_body,
    out_type=jax.ShapeDtypeStruct((NW, N), jnp.float32),
    mesh=_sc_mesh,
    compiler_params=pltpu.CompilerParams(needs_layout_passes=False, use_tc_tiling_on_sc=False),
    scratch_types=[
        pltpu.VMEM((EPW,), jnp.int32),
        pltpu.VMEM((EPW,), jnp.float32),
        pltpu.VMEM((N,), jnp.float32),
    ],
)


def _prep_body(degp_ref, x_ref, y_ref, dinv_ref):
    deg = jnp.sum(degp_ref[...], axis=0) + 1.0
    dinv = lax.rsqrt(deg)[:, None]
    y_ref[...] = x_ref[...] * dinv
    dinv_ref[...] = dinv


_prep_call = pl.pallas_call(
    _prep_body,
    grid=(GRID,),
    in_specs=[
        pl.BlockSpec((NW, BN), lambda i: (0, i)),
        pl.BlockSpec((BN, D), lambda i: (i, 0)),
    ],
    out_specs=[
        pl.BlockSpec((BN, D), lambda i: (i, 0)),
        pl.BlockSpec((BN, 1), lambda i: (i, 0)),
    ],
    out_shape=[
        jax.ShapeDtypeStruct((N, D), jnp.float32),
        jax.ShapeDtypeStruct((N, 1), jnp.float32),
    ],
)


NBUF = 4
NFULL = NCHUNK - (NCHUNK % NBUF)


def _edge_body(src_hbm, dstr_hbm, ew_hbm, y_hbm, accp_hbm,
               acc_sh, srcb, dstb, ewb, r0, r1, r2, r3,
               gsem, ssem, isem, jsem):
    rows = (r0, r1, r2, r3)
    sbuf = rows
    cid = lax.axis_index("c")
    sid = lax.axis_index("s")
    wid = cid * NS + sid
    rowbase = sid * RPT

    def zb(i, _):
        for k in range(8):
            r0[i, pl.ds(k * 16, 16)] = jnp.zeros((16,), jnp.float32)
        return 0

    lax.fori_loop(0, CE, zb, 0)

    def zcopy(j, _):
        pltpu.sync_copy(r0, acc_sh.at[pl.ds(rowbase + j * CE, CE)])
        return 0

    lax.fori_loop(0, RPT // CE, zcopy, 0)
    pltpu.sync_copy(r0.at[pl.ds(0, RPT % CE)],
                    acc_sh.at[pl.ds(rowbase + (RPT // CE) * CE, RPT % CE)])

    def start_src(c, b):
        pltpu.async_copy(src_hbm.at[pl.ds(wid * EPW + c * CE, CE)],
                         srcb.at[b], jsem.at[b])

    def wait_src(b):
        pltpu.make_async_copy(src_hbm.at[pl.ds(0, CE)], srcb.at[b],
                              jsem.at[b]).wait()

    def start_gather(c, b):
        pltpu.async_copy(y_hbm.at[srcb.at[b]], rows[b], gsem.at[b])

    def start_idx(c, b):
        pltpu.async_copy(dstr_hbm.at[wid * NCHUNK + c], dstb.at[b],
                         isem.at[b])
        pltpu.async_copy(ew_hbm.at[pl.ds(wid * EPW + c * CE, CE)],
                         ewb.at[b], isem.at[b])

    def wait_idx(b):
        pltpu.make_async_copy(dstr_hbm.at[0], dstb.at[b], isem.at[b]).wait()
        pltpu.make_async_copy(ew_hbm.at[pl.ds(0, CE)], ewb.at[b],
                              isem.at[b]).wait()

    def wait_gather(b):
        pltpu.make_async_copy(y_hbm.at[srcb.at[0]],
                              rows[b], gsem.at[b]).wait()

    def scale(b):
        def sg(g2, _):
            wv = ewb[b, pl.ds(g2 * 16, 16)]
            for j in range(16):
                w = wv[j]
                e = g2 * 16 + j
                for k in range(8):
                    rows[b][e, pl.ds(k * 16, 16)] = (
                        rows[b][e, pl.ds(k * 16, 16)] * w)
            return 0

        lax.fori_loop(0, CE // 16, sg, 0)

    def start_scatter(b):
        pltpu.async_copy(sbuf[b], acc_sh.at[dstb.at[b]], ssem.at[b],
                         add=True)

    def wait_scatter(b):
        pltpu.make_async_copy(sbuf[b], acc_sh.at[dstb.at[0]],
                              ssem.at[b]).wait()

    for b in range(NBUF):
        start_src(b, b)
        start_idx(b, b)
        wait_src(b)
        start_gather(b, b)
    plsc.subcore_barrier()

    def step(c, b):
        wait_gather(b)
        @pl.when(c + NBUF < NCHUNK)
        def _(c=c, b=b):
            start_src(c + NBUF, b)
        wait_idx(b)
        scale(b)
        start_scatter(b)

    def refill(c2, b):
        wait_scatter(b)

        @pl.when(c2 < NCHUNK)
        def _(c2=c2, b=b):
            wait_src(b)
            start_idx(c2, b)
            start_gather(c2, b)

    def outer(g, _):
        c0 = g * NBUF
        for b in range(NBUF):
            step(c0 + b, b)
        for b in range(NBUF):
            refill(c0 + b + NBUF, b)
        return 0

    lax.fori_loop(0, NFULL // NBUF, outer, 0)
    for i in range(NCHUNK % NBUF):
        step(NFULL + i, i)
    for i in range(NCHUNK % NBUF):
        wait_scatter(i)

    plsc.subcore_barrier()
    pltpu.sync_copy(acc_sh.at[pl.ds(rowbase, RPT)],
                    accp_hbm.at[cid, pl.ds(rowbase, RPT)])


_edge_call = pl.kernel(
    _edge_body,
    out_type=jax.ShapeDtypeStruct((NC, N, D), jnp.float32),
    mesh=_sc_mesh,
    compiler_params=pltpu.CompilerParams(needs_layout_passes=False, use_tc_tiling_on_sc=False),
    scratch_types=[
        pltpu.VMEM_SHARED((N, D), jnp.float32),
        pltpu.VMEM((NBUF, CE), jnp.int32),
        pltpu.VMEM((NBUF, CE), jnp.int32),
        pltpu.VMEM((NBUF, CE), jnp.float32),
        pltpu.VMEM((CE, D), jnp.float32),
        pltpu.VMEM((CE, D), jnp.float32),
        pltpu.VMEM((CE, D), jnp.float32),
        pltpu.VMEM((CE, D), jnp.float32),
        pltpu.SemaphoreType.DMA((NBUF,)),
        pltpu.SemaphoreType.DMA((NBUF,)),
        pltpu.SemaphoreType.DMA((NBUF,)),
        pltpu.SemaphoreType.DMA((NBUF,)),
    ],
)


def _out_body(acc_ref, x_ref, dinv_ref, mz_ref, mh_ref, wh_ref,
              cz_ref, ch_ref, bh_ref, o_ref):
    dv = dinv_ref[...]
    p = dv * (acc_ref[0] + acc_ref[1]) + dv * dv * x_ref[...]
    z = jax.nn.sigmoid(
        jnp.dot(p, mz_ref[...], preferred_element_type=jnp.float32)
        + cz_ref[...])
    t = (jnp.tanh(jnp.dot(p, mh_ref[...], preferred_element_type=jnp.float32)
                  + ch_ref[...])
         + jnp.dot(p, wh_ref[...], preferred_element_type=jnp.float32)
         + bh_ref[...])
    o_ref[...] = (1.0 - z) * t


_full = lambda i: (0, 0)
_out_call = pl.pallas_call(
    _out_body,
    grid=(GRID,),
    in_specs=[
        pl.BlockSpec((NC, BN, D), lambda i: (0, i, 0)),
        pl.BlockSpec((BN, D), lambda i: (i, 0)),
        pl.BlockSpec((BN, 1), lambda i: (i, 0)),
        pl.BlockSpec((D, D), _full),
        pl.BlockSpec((D, D), _full),
        pl.BlockSpec((D, D), _full),
        pl.BlockSpec((1, D), _full),
        pl.BlockSpec((1, D), _full),
        pl.BlockSpec((1, D), _full),
    ],
    out_specs=pl.BlockSpec((BN, D), lambda i: (i, 0)),
    out_shape=jax.ShapeDtypeStruct((N, D), jnp.float32),
)


@jax.jit
def kernel(X, edge_index, edge_weight, Wz, bz, Wr, br, Wh, bh,
           Lzw, Lzb, Lrw, Lrb, Lhw, Lhb):
    src = edge_index[0]
    dst = edge_index[1]
    eye = jnp.eye(D, dtype=jnp.float32)
    Lz = Lzw[:D]
    Lh = Lhw[:D]
    Mz = Wz @ (Lz + eye)
    cz = (bz @ Lz + bz + Lzb)[None, :]
    Mh = Wh @ Lh
    ch = (bh @ Lh + Lhb)[None, :]
    bh2 = bh[None, :]

    degp = _deg_call(dst, edge_weight)
    Y, dinv = _prep_call(degp, X)
    dstr = dst.reshape(E // CE, CE)
    accp = _edge_call(src, dstr, edge_weight, Y)
    return _out_call(accp, X, dinv, Mz, Mh, Wh, cz, ch, bh2)


# speedup vs baseline: 66.3444x; 1.1031x over previous
import functools

import jax
import jax.numpy as jnp
from jax import lax
from jax.experimental import pallas as pl
from jax.experimental.pallas import tpu as pltpu
from jax.experimental.pallas import tpu_sc as plsc

N = 10000
E = 320000
D = 128

NC = 2
NS = 16
NW = NC * NS
EPW = E // NW
CE = 80
NCHUNK = EPW // CE
RPT = N // NS
ZR = 25

BN = 1024
GRID = (N + BN - 1) // BN

_sc_mesh = plsc.VectorSubcoreMesh(core_axis_name="c", subcore_axis_name="s")


def _deg_body(dst_hbm, ew_hbm, degp_hbm, dst_v, ew_v, deg_v):
    cid = lax.axis_index("c")
    sid = lax.axis_index("s")
    wid = cid * NS + sid
    base = wid * EPW

    def zbody(i, _):
        deg_v[pl.ds(i * 16, 16)] = jnp.zeros((16,), jnp.float32)
        return 0

    lax.fori_loop(0, N // 16, zbody, 0)

    pltpu.sync_copy(dst_hbm.at[pl.ds(base, EPW)], dst_v)
    pltpu.sync_copy(ew_hbm.at[pl.ds(base, EPW)], ew_v)

    def ebody(i, _):
        idx = dst_v[pl.ds(i * 16, 16)]
        w = ew_v[pl.ds(i * 16, 16)]
        plsc.addupdate_scatter(deg_v, [idx], w)
        return 0

    lax.fori_loop(0, EPW // 16, ebody, 0)
    pltpu.sync_copy(deg_v, degp_hbm.at[wid])


_deg_call = pl.kernel(
    _deg_body,
    out_type=jax.ShapeDtypeStruct((NW, N), jnp.float32),
    mesh=_sc_mesh,
    compiler_params=pltpu.CompilerParams(needs_layout_passes=False, use_tc_tiling_on_sc=False),
    scratch_types=[
        pltpu.VMEM((EPW,), jnp.int32),
        pltpu.VMEM((EPW,), jnp.float32),
        pltpu.VMEM((N,), jnp.float32),
    ],
)


def _prep_body(degp_ref, x_ref, y_ref, dinv_ref):
    deg = jnp.sum(degp_ref[...], axis=0) + 1.0
    dinv = lax.rsqrt(deg)[:, None]
    y_ref[...] = x_ref[...] * dinv
    dinv_ref[...] = dinv


_prep_call = pl.pallas_call(
    _prep_body,
    grid=(GRID,),
    in_specs=[
        pl.BlockSpec((NW, BN), lambda i: (0, i)),
        pl.BlockSpec((BN, D), lambda i: (i, 0)),
    ],
    out_specs=[
        pl.BlockSpec((BN, D), lambda i: (i, 0)),
        pl.BlockSpec((BN, 1), lambda i: (i, 0)),
    ],
    out_shape=[
        jax.ShapeDtypeStruct((N, D), jnp.float32),
        jax.ShapeDtypeStruct((N, 1), jnp.float32),
    ],
)


NBUF = 4
NFULL = NCHUNK - (NCHUNK % NBUF)


def _edge_body(src_hbm, dstr_hbm, ew_hbm, y_hbm, accp_hbm,
               acc_sh, srcb, dstb, ewb, r0, r1, r2, r3,
               gsem, ssem, isem, jsem):
    rows = (r0, r1, r2, r3)
    sbuf = rows
    cid = lax.axis_index("c")
    sid = lax.axis_index("s")
    wid = cid * NS + sid
    rowbase = sid * RPT

    def zb(i, _):
        for k in range(8):
            r0[i, pl.ds(k * 16, 16)] = jnp.zeros((16,), jnp.float32)
        return 0

    lax.fori_loop(0, CE, zb, 0)

    def zcopy(j, _):
        pltpu.sync_copy(r0, acc_sh.at[pl.ds(rowbase + j * CE, CE)])
        return 0

    lax.fori_loop(0, RPT // CE, zcopy, 0)
    pltpu.sync_copy(r0.at[pl.ds(0, RPT % CE)],
                    acc_sh.at[pl.ds(rowbase + (RPT // CE) * CE, RPT % CE)])

    def start_src(c, b):
        pltpu.async_copy(src_hbm.at[pl.ds(wid * EPW + c * CE, CE)],
                         srcb.at[b], jsem.at[b])

    def wait_src(b):
        pltpu.make_async_copy(src_hbm.at[pl.ds(0, CE)], srcb.at[b],
                              jsem.at[b]).wait()

    def start_gather(c, b):
        pltpu.async_copy(y_hbm.at[srcb.at[b]], rows[b], gsem.at[b])

    def start_idx(c, b):
        pltpu.async_copy(dstr_hbm.at[wid * NCHUNK + c], dstb.at[b],
                         isem.at[b])
        pltpu.async_copy(ew_hbm.at[pl.ds(wid * EPW + c * CE, CE)],
                         ewb.at[b], isem.at[b])

    def wait_idx(b):
        pltpu.make_async_copy(dstr_hbm.at[0], dstb.at[b], isem.at[b]).wait()
        pltpu.make_async_copy(ew_hbm.at[pl.ds(0, CE)], ewb.at[b],
                              isem.at[b]).wait()

    def wait_gather(b):
        pltpu.make_async_copy(y_hbm.at[srcb.at[0]],
                              rows[b], gsem.at[b]).wait()

    def scale(b):
        def sg(g2, _):
            wv = ewb[b, pl.ds(g2 * 16, 16)]
            for j in range(16):
                w = wv[j]
                e = g2 * 16 + j
                for k in range(8):
                    rows[b][e, pl.ds(k * 16, 16)] = (
                        rows[b][e, pl.ds(k * 16, 16)] * w)
            return 0

        lax.fori_loop(0, CE // 16, sg, 0)

    def start_scatter(b):
        pltpu.async_copy(sbuf[b], acc_sh.at[dstb.at[b]], ssem.at[b],
                         add=True)

    def wait_scatter(b):
        pltpu.make_async_copy(sbuf[b], acc_sh.at[dstb.at[0]],
                              ssem.at[b]).wait()

    for b in range(NBUF):
        start_src(b, b)
        start_idx(b, b)
        wait_src(b)
        start_gather(b, b)
    plsc.subcore_barrier()

    def refill(cr, b1, guard):
        @pl.when(guard)
        def _():
            wait_scatter(b1)

        @pl.when(guard & (cr < NCHUNK))
        def _():
            wait_src(b1)
            start_idx(cr, b1)
            start_gather(cr, b1)

    def outer(g, _):
        c0 = g * NBUF
        for b in range(NBUF):
            c = c0 + b
            wait_gather(b)

            @pl.when(c + NBUF < NCHUNK)
            def _(c=c, b=b):
                start_src(c + NBUF, b)

            wait_idx(b)
            scale(b)
            start_scatter(b)
            guard = (g >= 1) if b == 0 else (g >= 0)
            refill(c - 1 + NBUF, (b - 1) % NBUF, guard)
        return 0

    lax.fori_loop(0, NFULL // NBUF, outer, 0)
    for i in range(NCHUNK % NBUF):
        c = NFULL + i
        wait_gather(i)
        wait_idx(i)
        scale(i)
        start_scatter(i)
        wait_scatter((i - 1) % NBUF)
    for i in range(NCHUNK % NBUF):
        wait_scatter(i)

    plsc.subcore_barrier()
    pltpu.sync_copy(acc_sh.at[pl.ds(rowbase, RPT)],
                    accp_hbm.at[cid, pl.ds(rowbase, RPT)])


_edge_call = pl.kernel(
    _edge_body,
    out_type=jax.ShapeDtypeStruct((NC, N, D), jnp.float32),
    mesh=_sc_mesh,
    compiler_params=pltpu.CompilerParams(needs_layout_passes=False, use_tc_tiling_on_sc=False),
    scratch_types=[
        pltpu.VMEM_SHARED((N, D), jnp.float32),
        pltpu.VMEM((NBUF, CE), jnp.int32),
        pltpu.VMEM((NBUF, CE), jnp.int32),
        pltpu.VMEM((NBUF, CE), jnp.float32),
        pltpu.VMEM((CE, D), jnp.float32),
        pltpu.VMEM((CE, D), jnp.float32),
        pltpu.VMEM((CE, D), jnp.float32),
        pltpu.VMEM((CE, D), jnp.float32),
        pltpu.SemaphoreType.DMA((NBUF,)),
        pltpu.SemaphoreType.DMA((NBUF,)),
        pltpu.SemaphoreType.DMA((NBUF,)),
        pltpu.SemaphoreType.DMA((NBUF,)),
    ],
)


def _out_body(acc_ref, x_ref, dinv_ref, mz_ref, mh_ref, wh_ref,
              cz_ref, ch_ref, bh_ref, o_ref):
    dv = dinv_ref[...]
    p = dv * (acc_ref[0] + acc_ref[1]) + dv * dv * x_ref[...]
    z = jax.nn.sigmoid(
        jnp.dot(p, mz_ref[...], preferred_element_type=jnp.float32)
        + cz_ref[...])
    t = (jnp.tanh(jnp.dot(p, mh_ref[...], preferred_element_type=jnp.float32)
                  + ch_ref[...])
         + jnp.dot(p, wh_ref[...], preferred_element_type=jnp.float32)
         + bh_ref[...])
    o_ref[...] = (1.0 - z) * t


_full = lambda i: (0, 0)
_out_call = pl.pallas_call(
    _out_body,
    grid=(GRID,),
    in_specs=[
        pl.BlockSpec((NC, BN, D), lambda i: (0, i, 0)),
        pl.BlockSpec((BN, D), lambda i: (i, 0)),
        pl.BlockSpec((BN, 1), lambda i: (i, 0)),
        pl.BlockSpec((D, D), _full),
        pl.BlockSpec((D, D), _full),
        pl.BlockSpec((D, D), _full),
        pl.BlockSpec((1, D), _full),
        pl.BlockSpec((1, D), _full),
        pl.BlockSpec((1, D), _full),
    ],
    out_specs=pl.BlockSpec((BN, D), lambda i: (i, 0)),
    out_shape=jax.ShapeDtypeStruct((N, D), jnp.float32),
)


@jax.jit
def kernel(X, edge_index, edge_weight, Wz, bz, Wr, br, Wh, bh,
           Lzw, Lzb, Lrw, Lrb, Lhw, Lhb):
    src = edge_index[0]
    dst = edge_index[1]
    eye = jnp.eye(D, dtype=jnp.float32)
    Lz = Lzw[:D]
    Lh = Lhw[:D]
    Mz = Wz @ (Lz + eye)
    cz = (bz @ Lz + bz + Lzb)[None, :]
    Mh = Wh @ Lh
    ch = (bh @ Lh + Lhb)[None, :]
    bh2 = bh[None, :]

    degp = _deg_call(dst, edge_weight)
    Y, dinv = _prep_call(degp, X)
    dstr = dst.reshape(E // CE, CE)
    accp = _edge_call(src, dstr, edge_weight, Y)
    return _out_call(accp, X, dinv, Mz, Mh, Wh, cz, ch, bh2)
